# Initial kernel scaffold; baseline (speedup 1.0000x reference)
import jax
import jax.numpy as jnp
from jax.experimental import pallas as pl


def kernel(x, down_0_w, down_0_b, res_0_0_w3, res_0_0_b3, res_0_0_w1, res_0_0_b1, res_0_1_w3, res_0_1_b3, res_0_1_w1, res_0_1_b1, res_0_2_w3, res_0_2_b3, res_0_2_w1, res_0_2_b1, res_0_3_w3, res_0_3_b3, res_0_3_w1, res_0_3_b1, down_1_w, down_1_b, res_1_0_w3, res_1_0_b3, res_1_0_w1, res_1_0_b1, res_1_1_w3, res_1_1_b3, res_1_1_w1, res_1_1_b1, res_1_2_w3, res_1_2_b3, res_1_2_w1, res_1_2_b1, res_1_3_w3, res_1_3_b3, res_1_3_w1, res_1_3_b1, out_w, out_b):
    raise NotImplementedError("write your pallas kernel here")


# speedup vs baseline: 5.2351x; 5.2351x over previous
import functools

import jax
import jax.numpy as jnp
from jax.experimental import pallas as pl
from jax.experimental.pallas import tpu as pltpu

_VMEM_LIMIT_BYTES = 100 * 1024 * 1024
_C = 64
_F = 4
_H = 8
_MAX_CHUNK = 2048


def _fold_bias(b, fold):
    return jnp.tile(b.astype(jnp.float32), fold).reshape(1, fold * b.shape[0])


def _same_conv_mats(w, dilation, fold):
    K, cin, cout = w.shape
    half = K // 2
    mats = {}
    for j in range(fold):
        for k in range(K):
            u = j + (k - half) * dilation
            r, ui = u // fold, u % fold
            if r not in mats:
                mats[r] = jnp.zeros((fold * cin, fold * cout), jnp.float32)
            mats[r] = mats[r].at[ui * cin:(ui + 1) * cin,
                                 j * cout:(j + 1) * cout].add(w[k])
    shifts = sorted(mats)
    return shifts, [mats[r].astype(jnp.bfloat16) for r in shifts]


def _down_conv_mats(w, fold_out):
    K, cin, cout = w.shape
    fold_in = 2 * fold_out
    mats = {}
    for j in range(fold_out):
        for k in range(K):
            u = 2 * j - 1 + k
            r, ui = u // fold_in, u % fold_in
            if r not in mats:
                mats[r] = jnp.zeros((fold_in * cin, fold_out * cout), jnp.float32)
            mats[r] = mats[r].at[ui * cin:(ui + 1) * cin,
                                 j * cout:(j + 1) * cout].add(w[k])
    shifts = sorted(mats)
    return shifts, [mats[r].astype(jnp.bfloat16) for r in shifts]


def _chain_kernel(x_hbm, *args, ops, n_weights, chunk, n_chunks, lanes_in):
    w_refs = args[:n_weights]
    o_ref = args[n_weights]
    win, buf0, buf1, sem = args[n_weights + 1:]
    n = pl.program_id(0)
    c = pl.program_id(1)
    c0 = c * chunk
    wrows = chunk + 2 * _H
    lanes_mid = _F * _C

    mid = pltpu.make_async_copy(x_hbm.at[n, pl.ds(c0, chunk), :],
                                win.at[pl.ds(_H, chunk)], sem.at[0])
    mid.start()

    @pl.when(c != 0)
    def _():
        top = pltpu.make_async_copy(x_hbm.at[n, pl.ds(c0 - _H, _H), :],
                                    win.at[pl.ds(0, _H)], sem.at[1])
        top.start()
        top.wait()

    @pl.when(c == 0)
    def _():
        win[0:_H, :] = jnp.zeros((_H, lanes_in), win.dtype)

    @pl.when(c != n_chunks - 1)
    def _():
        bot = pltpu.make_async_copy(x_hbm.at[n, pl.ds(c0 + chunk, _H), :],
                                    win.at[pl.ds(_H + chunk, _H)], sem.at[2])
        bot.start()
        bot.wait()

    @pl.when(c == n_chunks - 1)
    def _():
        win[_H + chunk:wrows, :] = jnp.zeros((_H, lanes_in), win.dtype)

    mid.wait()

    for b in (buf0, buf1):
        b[0:2, :] = jnp.zeros((2, lanes_mid), b.dtype)
        b[wrows - 2:wrows, :] = jnp.zeros((2, lanes_mid), b.dtype)

    lo, hi = 2, wrows - 2

    def conv_acc(src_ref, mat_refs, shifts, bias_ref, relu_in):
        zf = src_ref[...]
        if relu_in:
            zf = jnp.maximum(zf, 0.0)
        z = zf.astype(jnp.bfloat16)
        acc = None
        for r, mref in zip(shifts, mat_refs):
            part = jnp.dot(z[lo + r:hi + r, :], mref[...],
                           preferred_element_type=jnp.float32)
            acc = part if acc is None else acc + part
        return acc + bias_ref[...].astype(jnp.float32)

    def edge_zero(dst):
        @pl.when(c == 0)
        def _():
            dst[0:_H, :] = jnp.zeros((_H, lanes_mid), dst.dtype)

        @pl.when(c == n_chunks - 1)
        def _():
            dst[_H + chunk:wrows, :] = jnp.zeros((_H, lanes_mid), dst.dtype)

    src = win
    bufs = [buf0, buf1]
    bi = 0
    wi = 0
    for kind, shifts in ops:
        if kind == 'down':
            mats = w_refs[wi:wi + len(shifts)]
            bias = w_refs[wi + len(shifts)]
            wi += len(shifts) + 1
            dst = bufs[bi]
            bi ^= 1
            dst[lo:hi, :] = conv_acc(src, mats, shifts, bias, False)
            edge_zero(dst)
            src = dst
        elif kind == 'res':
            mats = w_refs[wi:wi + len(shifts)]
            b3 = w_refs[wi + len(shifts)]
            w1 = w_refs[wi + len(shifts) + 1]
            b1 = w_refs[wi + len(shifts) + 2]
            wi += len(shifts) + 3
            dst = bufs[bi]
            bi ^= 1
            acc = conv_acc(src, mats, shifts, b3, True)
            h2 = jnp.maximum(acc, 0.0).astype(jnp.bfloat16)
            y = jnp.dot(h2, w1[...], preferred_element_type=jnp.float32)
            y = y + b1[...].astype(jnp.float32)
            dst[lo:hi, :] = src[lo:hi, :] + y
            edge_zero(dst)
            src = dst
        else:
            mats = w_refs[wi:wi + len(shifts)]
            bias = w_refs[wi + len(shifts)]
            wi += len(shifts) + 1
            acc = conv_acc(src, mats, shifts, bias, False)
            o_ref[0] = acc[_H - lo:_H - lo + chunk, :]
            return
    o_ref[0] = src[_H:_H + chunk, :]


def _run_stage(x_folded, weights, ops, n_rows_out):
    n_batch = x_folded.shape[0]
    lanes_in = x_folded.shape[2]
    chunk = min(_MAX_CHUNK, n_rows_out)
    n_chunks = n_rows_out // chunk
    wrows = chunk + 2 * _H
    body = functools.partial(_chain_kernel, ops=ops, n_weights=len(weights),
                             chunk=chunk, n_chunks=n_chunks, lanes_in=lanes_in)
    in_specs = [pl.BlockSpec(memory_space=pl.ANY)]
    for w in weights:
        in_specs.append(pl.BlockSpec(w.shape, lambda n, c: (0, 0)))
    return pl.pallas_call(
        body,
        out_shape=jax.ShapeDtypeStruct((n_batch, n_rows_out, _F * _C),
                                       jnp.float32),
        grid=(n_batch, n_chunks),
        in_specs=in_specs,
        out_specs=pl.BlockSpec((1, chunk, _F * _C), lambda n, c: (n, c, 0)),
        scratch_shapes=[pltpu.VMEM((wrows, lanes_in), jnp.float32),
                        pltpu.VMEM((wrows, _F * _C), jnp.float32),
                        pltpu.VMEM((wrows, _F * _C), jnp.float32),
                        pltpu.SemaphoreType.DMA((3,))],
        compiler_params=pltpu.CompilerParams(
            dimension_semantics=("parallel", "parallel"),
            vmem_limit_bytes=_VMEM_LIMIT_BYTES),
    )(x_folded, *weights)


def _stage_weights(down_w, down_b, res_params, dilations, final_w=None,
                   final_b=None):
    weights = []
    ops = []
    shifts, mats = _down_conv_mats(down_w, _F)
    weights += mats + [_fold_bias(down_b, _F)]
    ops.append(('down', tuple(shifts)))
    for (w3, b3, w1, b1), d in zip(res_params, dilations):
        shifts, mats = _same_conv_mats(w3, d, _F)
        w1f = jnp.kron(jnp.eye(_F, dtype=jnp.float32),
                       w1[0]).astype(jnp.bfloat16)
        weights += mats + [_fold_bias(b3, _F), w1f, _fold_bias(b1, _F)]
        ops.append(('res', tuple(shifts)))
    if final_w is not None:
        shifts, mats = _same_conv_mats(final_w, 1, _F)
        weights += mats + [_fold_bias(final_b, _F)]
        ops.append(('final', tuple(shifts)))
    return weights, ops


def kernel(x, down_0_w, down_0_b, res_0_0_w3, res_0_0_b3, res_0_0_w1, res_0_0_b1, res_0_1_w3, res_0_1_b3, res_0_1_w1, res_0_1_b1, res_0_2_w3, res_0_2_b3, res_0_2_w1, res_0_2_b1, res_0_3_w3, res_0_3_b3, res_0_3_w1, res_0_3_b1, down_1_w, down_1_b, res_1_0_w3, res_1_0_b3, res_1_0_w1, res_1_0_b1, res_1_1_w3, res_1_1_b3, res_1_1_w1, res_1_1_b1, res_1_2_w3, res_1_2_b3, res_1_2_w1, res_1_2_b1, res_1_3_w3, res_1_3_b3, res_1_3_w1, res_1_3_b1, out_w, out_b):
    n_batch, t_len, c_in = x.shape
    dilations = (1, 2, 4, 8)

    x_folded = x.reshape(n_batch, t_len // (2 * _F), 2 * _F * c_in)
    w_a, ops_a = _stage_weights(
        down_0_w, down_0_b,
        [(res_0_0_w3, res_0_0_b3, res_0_0_w1, res_0_0_b1),
         (res_0_1_w3, res_0_1_b3, res_0_1_w1, res_0_1_b1),
         (res_0_2_w3, res_0_2_b3, res_0_2_w1, res_0_2_b1),
         (res_0_3_w3, res_0_3_b3, res_0_3_w1, res_0_3_b1)], dilations)
    h = _run_stage(x_folded, w_a, ops_a, t_len // (2 * _F))

    h_folded = h.reshape(n_batch, t_len // (4 * _F), 2 * _F * _C)
    w_b, ops_b = _stage_weights(
        down_1_w, down_1_b,
        [(res_1_0_w3, res_1_0_b3, res_1_0_w1, res_1_0_b1),
         (res_1_1_w3, res_1_1_b3, res_1_1_w1, res_1_1_b1),
         (res_1_2_w3, res_1_2_b3, res_1_2_w1, res_1_2_b1),
         (res_1_3_w3, res_1_3_b3, res_1_3_w1, res_1_3_b1)], dilations,
        final_w=out_w, final_b=out_b)
    out = _run_stage(h_folded, w_b, ops_b, t_len // (4 * _F))
    return out.reshape(n_batch, t_len // 4, _C)


# speedup vs baseline: 6.5292x; 1.2472x over previous
import functools

import jax
import jax.numpy as jnp
from jax.experimental import pallas as pl
from jax.experimental.pallas import tpu as pltpu

_VMEM_LIMIT_BYTES = 100 * 1024 * 1024
_C = 64
_F = 4
_H = 32
_LO = 16
_GUARD = 8
_MAX_CHUNK = 2048


def _fold_bias(b, fold):
    return jnp.tile(b.astype(jnp.float32), fold).reshape(1, fold * b.shape[0])


def _pack_blocks(blocks, fold_in, cin, cout, fold_out):
    out = []
    for r in sorted(blocks):
        us = [u for u, _, _ in blocks[r]]
        u_lo, u_hi = min(us), max(us) + 1
        m = jnp.zeros(((u_hi - u_lo) * cin, fold_out * cout), jnp.float32)
        for u, j, w in blocks[r]:
            m = m.at[(u - u_lo) * cin:(u - u_lo + 1) * cin,
                     j * cout:(j + 1) * cout].add(w)
        out.append((r, u_lo, u_hi, m.astype(jnp.bfloat16)))
    return out


def _same_conv_mats(w, dilation, fold):
    K, cin, cout = w.shape
    blocks = {}
    for j in range(fold):
        for k in range(K):
            u = j + (k - K // 2) * dilation
            blocks.setdefault(u // fold, []).append((u % fold, j, w[k]))
    return _pack_blocks(blocks, fold, cin, cout, fold)


def _down_conv_mats(w, fold_out):
    K, cin, cout = w.shape
    fold_in = 2 * fold_out
    blocks = {}
    for j in range(fold_out):
        for k in range(K):
            u = 2 * j - 1 + k
            blocks.setdefault(u // fold_in, []).append((u % fold_in, j, w[k]))
    return _pack_blocks(blocks, fold_in, cin, cout, fold_out)


def _chain_kernel(x_hbm, *args, ops, n_weights, chunk, n_chunks, n_batch,
                  lanes_in, out_f32):
    w_refs = args[:n_weights]
    o_ref = args[n_weights]
    wins, buf0, buf1, sem = args[n_weights + 1:]
    n = pl.program_id(0)
    c = pl.program_id(1)
    g = n * n_chunks + c
    slot = jax.lax.rem(g, 2)
    wrows = chunk + 2 * _H
    lanes_mid = _F * _C
    cin = _C

    def dma_descs(nn, cc, sl):
        c0 = cc * chunk
        mid = pltpu.make_async_copy(
            x_hbm.at[nn, pl.ds(c0, chunk), :],
            wins.at[sl, pl.ds(_H, chunk), :], sem.at[sl, 0])
        top = pltpu.make_async_copy(
            x_hbm.at[nn, pl.ds(c0 - _H, _H), :],
            wins.at[sl, pl.ds(0, _H), :], sem.at[sl, 1])
        bot = pltpu.make_async_copy(
            x_hbm.at[nn, pl.ds(c0 + chunk, _H), :],
            wins.at[sl, pl.ds(_H + chunk, _H), :], sem.at[sl, 2])
        return mid, top, bot

    def issue(nn, cc, sl):
        mid, top, bot = dma_descs(nn, cc, sl)
        mid.start()

        @pl.when(cc != 0)
        def _():
            top.start()

        @pl.when(cc != n_chunks - 1)
        def _():
            bot.start()

    @pl.when(g == 0)
    def _():
        issue(n, c, slot)

    @pl.when(g + 1 < n_batch * n_chunks)
    def _():
        g1 = g + 1
        issue(g1 // n_chunks, jax.lax.rem(g1, n_chunks), 1 - slot)

    mid, top, bot = dma_descs(n, c, slot)
    mid.wait()

    @pl.when(c != 0)
    def _():
        top.wait()

    @pl.when(c == 0)
    def _():
        wins[slot, 0:_H, :] = jnp.zeros((_H, lanes_in), wins.dtype)

    @pl.when(c != n_chunks - 1)
    def _():
        bot.wait()

    @pl.when(c == n_chunks - 1)
    def _():
        wins[slot, _H + chunk:wrows, :] = jnp.zeros((_H, lanes_in), wins.dtype)

    win = wins.at[slot]
    lo, hi = _LO, wrows - _LO

    for b in (buf0, buf1):
        b[lo - _GUARD:lo, :] = jnp.zeros((_GUARD, lanes_mid), b.dtype)
        b[hi:hi + _GUARD, :] = jnp.zeros((_GUARD, lanes_mid), b.dtype)

    def conv_acc(src, spans, mat_refs, bias_ref, relu_in):
        acc = None
        for (r, u_lo, u_hi), mref in zip(spans, mat_refs):
            sl = src[lo + r:hi + r, u_lo * cin:u_hi * cin]
            if relu_in:
                sl = jnp.maximum(sl, jnp.zeros((), sl.dtype))
            part = jnp.dot(sl, mref[...], preferred_element_type=jnp.float32)
            acc = part if acc is None else acc + part
        return acc + bias_ref[...].astype(jnp.float32)

    def edge_zero(dst):
        @pl.when(c == 0)
        def _():
            dst[0:_H, :] = jnp.zeros((_H, lanes_mid), dst.dtype)

        @pl.when(c == n_chunks - 1)
        def _():
            dst[_H + chunk:wrows, :] = jnp.zeros((_H, lanes_mid), dst.dtype)

    src = win
    bufs = [buf0, buf1]
    bi = 0
    wi = 0
    for kind, spans in ops:
        if kind == 'down':
            mats = w_refs[wi:wi + len(spans)]
            bias = w_refs[wi + len(spans)]
            wi += len(spans) + 1
            dst = bufs[bi]
            bi ^= 1
            acc = conv_acc(src, spans, mats, bias, False)
            dst[lo:hi, :] = acc.astype(dst.dtype)
            edge_zero(dst)
            src = dst
        elif kind == 'res':
            mats = w_refs[wi:wi + len(spans)]
            b3 = w_refs[wi + len(spans)]
            w1 = w_refs[wi + len(spans) + 1]
            b1 = w_refs[wi + len(spans) + 2]
            wi += len(spans) + 3
            dst = bufs[bi]
            bi ^= 1
            acc = conv_acc(src, spans, mats, b3, True)
            h2 = jnp.maximum(acc, 0.0).astype(jnp.bfloat16)
            y = jnp.dot(h2, w1[...], preferred_element_type=jnp.float32)
            y = y + b1[...].astype(jnp.float32)
            out = src[lo:hi, :].astype(jnp.float32) + y
            dst[lo:hi, :] = out.astype(dst.dtype)
            edge_zero(dst)
            src = dst
        else:
            mats = w_refs[wi:wi + len(spans)]
            bias = w_refs[wi + len(spans)]
            wi += len(spans) + 1
            acc = conv_acc(src, spans, mats, bias, False)
            out = acc[_H - lo:_H - lo + chunk, :]
            o_ref[0] = out if out_f32 else out.astype(o_ref.dtype)
            return
    o_ref[0] = src[_H:_H + chunk, :]


def _run_stage(x_folded, weights, ops, n_rows_out, out_dtype):
    n_batch = x_folded.shape[0]
    lanes_in = x_folded.shape[2]
    chunk = min(_MAX_CHUNK, n_rows_out)
    n_chunks = n_rows_out // chunk
    wrows = chunk + 2 * _H
    body = functools.partial(_chain_kernel, ops=ops, n_weights=len(weights),
                             chunk=chunk, n_chunks=n_chunks, n_batch=n_batch,
                             lanes_in=lanes_in,
                             out_f32=(out_dtype == jnp.float32))
    in_specs = [pl.BlockSpec(memory_space=pl.ANY)]
    for w in weights:
        in_specs.append(pl.BlockSpec(w.shape, lambda n, c: (0, 0)))
    return pl.pallas_call(
        body,
        out_shape=jax.ShapeDtypeStruct((n_batch, n_rows_out, _F * _C),
                                       out_dtype),
        grid=(n_batch, n_chunks),
        in_specs=in_specs,
        out_specs=pl.BlockSpec((1, chunk, _F * _C), lambda n, c: (n, c, 0)),
        scratch_shapes=[pltpu.VMEM((2, wrows, lanes_in), jnp.bfloat16),
                        pltpu.VMEM((wrows, _F * _C), jnp.bfloat16),
                        pltpu.VMEM((wrows, _F * _C), jnp.bfloat16),
                        pltpu.SemaphoreType.DMA((2, 3))],
        compiler_params=pltpu.CompilerParams(
            dimension_semantics=("arbitrary", "arbitrary"),
            vmem_limit_bytes=_VMEM_LIMIT_BYTES),
    )(x_folded, *weights)


def _stage_weights(down_w, down_b, res_params, dilations, final_w=None,
                   final_b=None):
    weights = []
    ops = []

    def add(packed):
        spans = tuple((r, u_lo, u_hi) for r, u_lo, u_hi, _ in packed)
        weights.extend(m for _, _, _, m in packed)
        return spans

    spans = add(_down_conv_mats(down_w, _F))
    weights.append(_fold_bias(down_b, _F))
    ops.append(('down', spans))
    for (w3, b3, w1, b1), d in zip(res_params, dilations):
        spans = add(_same_conv_mats(w3, d, _F))
        w1f = jnp.kron(jnp.eye(_F, dtype=jnp.float32),
                       w1[0]).astype(jnp.bfloat16)
        weights += [_fold_bias(b3, _F), w1f, _fold_bias(b1, _F)]
        ops.append(('res', spans))
    if final_w is not None:
        spans = add(_same_conv_mats(final_w, 1, _F))
        weights.append(_fold_bias(final_b, _F))
        ops.append(('final', spans))
    return weights, ops


def kernel(x, down_0_w, down_0_b, res_0_0_w3, res_0_0_b3, res_0_0_w1, res_0_0_b1, res_0_1_w3, res_0_1_b3, res_0_1_w1, res_0_1_b1, res_0_2_w3, res_0_2_b3, res_0_2_w1, res_0_2_b1, res_0_3_w3, res_0_3_b3, res_0_3_w1, res_0_3_b1, down_1_w, down_1_b, res_1_0_w3, res_1_0_b3, res_1_0_w1, res_1_0_b1, res_1_1_w3, res_1_1_b3, res_1_1_w1, res_1_1_b1, res_1_2_w3, res_1_2_b3, res_1_2_w1, res_1_2_b1, res_1_3_w3, res_1_3_b3, res_1_3_w1, res_1_3_b1, out_w, out_b):
    n_batch, t_len, c_in = x.shape
    dilations = (1, 2, 4, 8)

    x_folded = x.reshape(n_batch, t_len // (2 * _F),
                         2 * _F * c_in).astype(jnp.bfloat16)
    w_a, ops_a = _stage_weights(
        down_0_w, down_0_b,
        [(res_0_0_w3, res_0_0_b3, res_0_0_w1, res_0_0_b1),
         (res_0_1_w3, res_0_1_b3, res_0_1_w1, res_0_1_b1),
         (res_0_2_w3, res_0_2_b3, res_0_2_w1, res_0_2_b1),
         (res_0_3_w3, res_0_3_b3, res_0_3_w1, res_0_3_b1)], dilations)
    h = _run_stage(x_folded, w_a, ops_a, t_len // (2 * _F), jnp.bfloat16)

    h_folded = h.reshape(n_batch, t_len // (4 * _F), 2 * _F * _C)
    w_b, ops_b = _stage_weights(
        down_1_w, down_1_b,
        [(res_1_0_w3, res_1_0_b3, res_1_0_w1, res_1_0_b1),
         (res_1_1_w3, res_1_1_b3, res_1_1_w1, res_1_1_b1),
         (res_1_2_w3, res_1_2_b3, res_1_2_w1, res_1_2_b1),
         (res_1_3_w3, res_1_3_b3, res_1_3_w1, res_1_3_b1)], dilations,
        final_w=out_w, final_b=out_b)
    out = _run_stage(h_folded, w_b, ops_b, t_len // (4 * _F), jnp.float32)
    return out.reshape(n_batch, t_len // 4, _C)


# speedup vs baseline: 7.3986x; 1.1332x over previous
import functools

import numpy as np
import jax
import jax.numpy as jnp
from jax.experimental import pallas as pl
from jax.experimental.pallas import tpu as pltpu

_VMEM_LIMIT_BYTES = 100 * 1024 * 1024
_C = 64
_F = 4
_H = 32
_LO = 16
_GUARD = 8
_MAX_CHUNK = 2048


def _conv_pattern(K, dilation, fold_in, stride=1):
    fold_out = fold_in // (2 if stride == 2 else 1)
    hits = {}
    for j in range(fold_out):
        for k in range(K):
            u = 2 * j - 1 + k if stride == 2 else j + (k - K // 2) * dilation
            hits.setdefault(u // fold_in, []).append((k, u % fold_in, j))
    rs = sorted(hits)
    P = np.zeros((K, len(rs), fold_in, fold_out), np.float32)
    spans = []
    for ri, r in enumerate(rs):
        us = [u for _, u, _ in hits[r]]
        spans.append((r, min(us), max(us) + 1))
        for k, u, j in hits[r]:
            P[k, ri, u, j] = 1.0
    return P, spans


def _stage_arrays(down_w, down_b, res_params, dilations, final_wb):
    pd, spans_d = _conv_pattern(4, 1, 2 * _F, stride=2)
    down_mats = jnp.einsum('kruj,kio->ruijo', pd, down_w)
    down_mats = down_mats.reshape(3, 2 * _F * _C, _F * _C).astype(jnp.bfloat16)
    down_bias = _tile_bias(down_b[None])

    conv_ws = [w3 for w3, _, _, _ in res_params]
    conv_bs = [b3 for _, b3, _, _ in res_params]
    dils = list(dilations)
    if final_wb is not None:
        conv_ws.append(final_wb[0])
        conv_bs.append(final_wb[1])
        dils.append(1)
    pats, all_spans = [], []
    for d in dils:
        p, spans = _conv_pattern(3, d, _F)
        pats.append(p)
        all_spans.append(tuple(spans))
    pc = np.stack(pats)
    conv_mats = jnp.einsum('ckruj,ckio->cruijo', pc, jnp.stack(conv_ws))
    conv_mats = conv_mats.reshape(len(dils), 3, _F * _C,
                                  _F * _C).astype(jnp.bfloat16)
    conv_biases = _tile_bias(jnp.stack(conv_bs))

    eye = np.eye(_F, dtype=np.float32)
    w1_mats = jnp.einsum('uj,cio->cuijo',
                         eye, jnp.stack([w1[0] for _, _, w1, _ in res_params]))
    w1_mats = w1_mats.reshape(4, _F * _C, _F * _C).astype(jnp.bfloat16)
    b1_biases = _tile_bias(jnp.stack([b1 for _, _, _, b1 in res_params]))

    ops = [('down', None, tuple(spans_d))]
    for ci in range(4):
        ops.append(('res', ci, all_spans[ci]))
    if final_wb is not None:
        ops.append(('final', 4, all_spans[4]))
    arrays = [down_mats, down_bias, conv_mats, conv_biases, w1_mats, b1_biases]
    return arrays, ops


def _tile_bias(b):
    return jnp.tile(b.astype(jnp.float32), (1, _F))


def _chain_kernel(x_hbm, down_mats, down_bias, conv_mats, conv_biases,
                  w1_mats, b1_biases, o_ref, wins, buf0, buf1, sem, *,
                  ops, chunk, n_chunks, n_batch, lanes_in, out_f32):
    n = pl.program_id(0)
    c = pl.program_id(1)
    g = n * n_chunks + c
    slot = jax.lax.rem(g, 2)
    wrows = chunk + 2 * _H
    lanes_mid = _F * _C
    cin = _C

    def dma_descs(nn, cc, sl):
        c0 = cc * chunk
        mid = pltpu.make_async_copy(
            x_hbm.at[nn, pl.ds(c0, chunk), :],
            wins.at[sl, pl.ds(_H, chunk), :], sem.at[sl, 0])
        top = pltpu.make_async_copy(
            x_hbm.at[nn, pl.ds(c0 - _H, _H), :],
            wins.at[sl, pl.ds(0, _H), :], sem.at[sl, 1])
        bot = pltpu.make_async_copy(
            x_hbm.at[nn, pl.ds(c0 + chunk, _H), :],
            wins.at[sl, pl.ds(_H + chunk, _H), :], sem.at[sl, 2])
        return mid, top, bot

    def issue(nn, cc, sl):
        mid, top, bot = dma_descs(nn, cc, sl)
        mid.start()

        @pl.when(cc != 0)
        def _():
            top.start()

        @pl.when(cc != n_chunks - 1)
        def _():
            bot.start()

    @pl.when(g == 0)
    def _():
        issue(n, c, slot)

    @pl.when(g + 1 < n_batch * n_chunks)
    def _():
        g1 = g + 1
        issue(g1 // n_chunks, jax.lax.rem(g1, n_chunks), 1 - slot)

    mid, top, bot = dma_descs(n, c, slot)
    mid.wait()

    @pl.when(c != 0)
    def _():
        top.wait()

    @pl.when(c == 0)
    def _():
        wins[slot, 0:_H, :] = jnp.zeros((_H, lanes_in), wins.dtype)

    @pl.when(c != n_chunks - 1)
    def _():
        bot.wait()

    @pl.when(c == n_chunks - 1)
    def _():
        wins[slot, _H + chunk:wrows, :] = jnp.zeros((_H, lanes_in), wins.dtype)

    win = wins.at[slot]
    lo, hi = _LO, wrows - _LO

    for b in (buf0, buf1):
        b[lo - _GUARD:lo, :] = jnp.zeros((_GUARD, lanes_mid), b.dtype)
        b[hi:hi + _GUARD, :] = jnp.zeros((_GUARD, lanes_mid), b.dtype)

    def conv_acc(src, spans, get_mat, bias, relu_in):
        acc = None
        for ti, (r, u_lo, u_hi) in enumerate(spans):
            sl = src[lo + r:hi + r, u_lo * cin:u_hi * cin]
            if relu_in:
                sl = jnp.maximum(sl, jnp.zeros((), sl.dtype))
            part = jnp.dot(sl, get_mat(ti, u_lo, u_hi),
                           preferred_element_type=jnp.float32)
            acc = part if acc is None else acc + part
        return acc + bias.astype(jnp.float32)

    def edge_zero(dst):
        @pl.when(c == 0)
        def _():
            dst[0:_H, :] = jnp.zeros((_H, lanes_mid), dst.dtype)

        @pl.when(c == n_chunks - 1)
        def _():
            dst[_H + chunk:wrows, :] = jnp.zeros((_H, lanes_mid), dst.dtype)

    src = win
    bufs = [buf0, buf1]
    bi = 0
    for kind, ci, spans in ops:
        if kind == 'down':
            acc = conv_acc(
                src, spans,
                lambda ti, a, b: down_mats[ti, a * cin:b * cin, :],
                down_bias[0:1, :], False)
            dst = bufs[bi]
            bi ^= 1
            dst[lo:hi, :] = acc.astype(dst.dtype)
            edge_zero(dst)
            src = dst
        elif kind == 'res':
            acc = conv_acc(
                src, spans,
                lambda ti, a, b: conv_mats[ci, ti, a * cin:b * cin, :],
                conv_biases[ci:ci + 1, :], True)
            h2 = jnp.maximum(acc, 0.0).astype(jnp.bfloat16)
            y = jnp.dot(h2, w1_mats[ci], preferred_element_type=jnp.float32)
            y = y + b1_biases[ci:ci + 1, :].astype(jnp.float32)
            out = src[lo:hi, :].astype(jnp.float32) + y
            dst = bufs[bi]
            bi ^= 1
            dst[lo:hi, :] = out.astype(dst.dtype)
            edge_zero(dst)
            src = dst
        else:
            acc = conv_acc(
                src, spans,
                lambda ti, a, b: conv_mats[ci, ti, a * cin:b * cin, :],
                conv_biases[ci:ci + 1, :], False)
            out = acc[_H - lo:_H - lo + chunk, :]
            o_ref[0] = out if out_f32 else out.astype(o_ref.dtype)
            return
    o_ref[0] = src[_H:_H + chunk, :]


def _run_stage(x_folded, arrays, ops, n_rows_out, out_dtype):
    n_batch = x_folded.shape[0]
    lanes_in = x_folded.shape[2]
    chunk = min(_MAX_CHUNK, n_rows_out)
    n_chunks = n_rows_out // chunk
    wrows = chunk + 2 * _H
    body = functools.partial(_chain_kernel, ops=ops, chunk=chunk,
                             n_chunks=n_chunks, n_batch=n_batch,
                             lanes_in=lanes_in,
                             out_f32=(out_dtype == jnp.float32))
    in_specs = [pl.BlockSpec(memory_space=pl.ANY)]
    for w in arrays:
        in_specs.append(
            pl.BlockSpec(w.shape, lambda *_, nd=w.ndim: (0,) * nd))
    return pl.pallas_call(
        body,
        out_shape=jax.ShapeDtypeStruct((n_batch, n_rows_out, _F * _C),
                                       out_dtype),
        grid=(n_batch, n_chunks),
        in_specs=in_specs,
        out_specs=pl.BlockSpec((1, chunk, _F * _C), lambda n, c: (n, c, 0)),
        scratch_shapes=[pltpu.VMEM((2, wrows, lanes_in), jnp.bfloat16),
                        pltpu.VMEM((wrows, _F * _C), jnp.bfloat16),
                        pltpu.VMEM((wrows, _F * _C), jnp.bfloat16),
                        pltpu.SemaphoreType.DMA((2, 3))],
        compiler_params=pltpu.CompilerParams(
            dimension_semantics=("arbitrary", "arbitrary"),
            vmem_limit_bytes=_VMEM_LIMIT_BYTES),
    )(x_folded, *arrays)


def kernel(x, down_0_w, down_0_b, res_0_0_w3, res_0_0_b3, res_0_0_w1, res_0_0_b1, res_0_1_w3, res_0_1_b3, res_0_1_w1, res_0_1_b1, res_0_2_w3, res_0_2_b3, res_0_2_w1, res_0_2_b1, res_0_3_w3, res_0_3_b3, res_0_3_w1, res_0_3_b1, down_1_w, down_1_b, res_1_0_w3, res_1_0_b3, res_1_0_w1, res_1_0_b1, res_1_1_w3, res_1_1_b3, res_1_1_w1, res_1_1_b1, res_1_2_w3, res_1_2_b3, res_1_2_w1, res_1_2_b1, res_1_3_w3, res_1_3_b3, res_1_3_w1, res_1_3_b1, out_w, out_b):
    n_batch, t_len, c_in = x.shape
    dilations = (1, 2, 4, 8)

    x_folded = x.reshape(n_batch, t_len // (2 * _F),
                         2 * _F * c_in).astype(jnp.bfloat16)
    arrs_a, ops_a = _stage_arrays(
        down_0_w, down_0_b,
        [(res_0_0_w3, res_0_0_b3, res_0_0_w1, res_0_0_b1),
         (res_0_1_w3, res_0_1_b3, res_0_1_w1, res_0_1_b1),
         (res_0_2_w3, res_0_2_b3, res_0_2_w1, res_0_2_b1),
         (res_0_3_w3, res_0_3_b3, res_0_3_w1, res_0_3_b1)], dilations, None)
    h = _run_stage(x_folded, arrs_a, ops_a, t_len // (2 * _F), jnp.bfloat16)

    h_folded = h.reshape(n_batch, t_len // (4 * _F), 2 * _F * _C)
    arrs_b, ops_b = _stage_arrays(
        down_1_w, down_1_b,
        [(res_1_0_w3, res_1_0_b3, res_1_0_w1, res_1_0_b1),
         (res_1_1_w3, res_1_1_b3, res_1_1_w1, res_1_1_b1),
         (res_1_2_w3, res_1_2_b3, res_1_2_w1, res_1_2_b1),
         (res_1_3_w3, res_1_3_b3, res_1_3_w1, res_1_3_b1)], dilations,
        (out_w, out_b))
    out = _run_stage(h_folded, arrs_b, ops_b, t_len // (4 * _F), jnp.float32)
    return out.reshape(n_batch, t_len // 4, _C)
